# baseline (speedup 1.0000x reference)
import functools

import jax
import jax.numpy as jnp
from jax import lax
from jax.experimental import pallas as pl
from jax.experimental.pallas import tpu as pltpu
from jax.experimental.pallas import tpu_sc as plsc

NUM_HIDDEN = 32
NUM_ROWS = 1000000
BATCH = 16384
NC = 2
NS = 16
NW = NC * NS
B_PER_W = BATCH // NW
CH = 128
NCH = B_PER_W // CH
L = 16
PANELS = 4
SROWS = NUM_ROWS // PANELS
SWIDE = PANELS * NUM_HIDDEN

RL_GRID = 1250
RL_ROWS = SROWS // RL_GRID


def _relayout_body(in0, in1, in2, in3, out_ref):
    out_ref[...] = jnp.concatenate(
        [in0[...], in1[...], in2[...], in3[...]], axis=1)


def _make_in_spec(k):
    return pl.BlockSpec((RL_ROWS, NUM_HIDDEN),
                        lambda i, _k=k: (i + _k * RL_GRID, 0))


_TC_RELAYOUT = pl.pallas_call(
    _relayout_body,
    grid=(RL_GRID,),
    in_specs=[_make_in_spec(k) for k in range(PANELS)] * 1,
    out_specs=pl.BlockSpec((RL_ROWS, SWIDE), lambda i: (i, 0)),
    out_shape=jax.ShapeDtypeStruct((SROWS, SWIDE), jnp.float32),
)


def _make_sc_kernel():
    mesh = plsc.VectorSubcoreMesh(core_axis_name="c", subcore_axis_name="s")

    @functools.partial(
        pl.kernel,
        mesh=mesh,
        compiler_params=pltpu.CompilerParams(needs_layout_passes=False),
        out_type=jax.ShapeDtypeStruct((BATCH,), jnp.float32),
        scratch_types=[
            pltpu.VMEM((B_PER_W,), jnp.int32),
            pltpu.VMEM((B_PER_W,), jnp.int32),
            pltpu.VMEM((CH,), jnp.int32),
            pltpu.VMEM((CH,), jnp.int32),
            pltpu.VMEM((CH, SWIDE), jnp.float32),
            pltpu.VMEM((CH, SWIDE), jnp.float32),
            pltpu.VMEM((B_PER_W,), jnp.float32),
            pltpu.SemaphoreType.DMA,
            pltpu.SemaphoreType.DMA,
        ],
    )
    def sc_kernel(uidx_hbm, iidx_hbm, user_hbm, item_hbm, out_hbm,
                  uidx_v, iidx_v, utidx_v, itidx_v, urows_v, irows_v, out_v,
                  sem_u, sem_i):
        wid = lax.axis_index("s") * NC + lax.axis_index("c")
        base = wid * B_PER_W
        row_iota = lax.iota(jnp.int32, L)

        pltpu.sync_copy(uidx_hbm.at[pl.ds(base, B_PER_W)], uidx_v)
        pltpu.sync_copy(iidx_hbm.at[pl.ds(base, B_PER_W)], iidx_v)

        def panel(vec):
            k = ((vec >= SROWS).astype(jnp.int32)
                 + (vec >= 2 * SROWS).astype(jnp.int32)
                 + (vec >= 3 * SROWS).astype(jnp.int32))
            return k

        def chunk_body(c, carry):
            cb = c * CH

            def tidx_body(g, cc):
                uvec = uidx_v[pl.ds(cb + g * L, L)]
                ivec = iidx_v[pl.ds(cb + g * L, L)]
                utidx_v[pl.ds(g * L, L)] = uvec - panel(uvec) * SROWS
                itidx_v[pl.ds(g * L, L)] = ivec - panel(ivec) * SROWS
                return cc

            lax.fori_loop(0, CH // L, tidx_body, 0)

            cp_u = pltpu.async_copy(user_hbm.at[utidx_v], urows_v, sem_u)
            cp_i = pltpu.async_copy(item_hbm.at[itidx_v], irows_v, sem_i)
            cp_u.wait()
            cp_i.wait()

            def group_body(g, cc):
                pos = g * L + row_iota
                uvec = uidx_v[pl.ds(cb + g * L, L)]
                ivec = iidx_v[pl.ds(cb + g * L, L)]
                ju = panel(uvec) * NUM_HIDDEN
                ji = panel(ivec) * NUM_HIDDEN
                acc = jnp.zeros((L,), jnp.float32)
                for h in range(NUM_HIDDEN):
                    u = plsc.load_gather(urows_v, [pos, ju + h])
                    v = plsc.load_gather(irows_v, [pos, ji + h])
                    acc = acc + u * v
                out_v[pl.ds(cb + g * L, L)] = acc
                return cc

            lax.fori_loop(0, CH // L, group_body, 0)
            return carry

        lax.fori_loop(0, NCH, chunk_body, 0)
        pltpu.sync_copy(out_v, out_hbm.at[pl.ds(base, B_PER_W)])

    return sc_kernel


_SC_KERNEL = _make_sc_kernel()


@jax.jit
def kernel(indices, ratings, user_table, item_table):
    idx = indices.astype(jnp.int32)
    u2 = _TC_RELAYOUT(user_table, user_table, user_table, user_table)
    i2 = _TC_RELAYOUT(item_table, item_table, item_table, item_table)
    pred = _SC_KERNEL(idx[0], idx[1], u2, i2)
    return (pred, ratings)


# speedup vs baseline: 3.6024x; 3.6024x over previous
import functools

import jax
import jax.numpy as jnp
from jax import lax
from jax.experimental import pallas as pl
from jax.experimental.pallas import tpu as pltpu
from jax.experimental.pallas import tpu_sc as plsc

NUM_HIDDEN = 32
BATCH = 16384
NC = 2
NS = 16
NW = NC * NS
B_PER_W = BATCH // NW
HALF = B_PER_W // 2
L = 16


def _scalar(vec, j):
    return jnp.squeeze(lax.slice(vec, (j,), (j + 1,)))


def _make_sc_kernel():
    mesh = plsc.VectorSubcoreMesh(core_axis_name="c", subcore_axis_name="s")

    @functools.partial(
        pl.kernel,
        mesh=mesh,
        compiler_params=pltpu.CompilerParams(needs_layout_passes=False),
        out_type=jax.ShapeDtypeStruct((BATCH,), jnp.float32),
        scratch_types=[
            pltpu.VMEM((B_PER_W,), jnp.int32),
            pltpu.VMEM((B_PER_W,), jnp.int32),
            pltpu.VMEM((HALF, NUM_HIDDEN), jnp.float32),
            pltpu.VMEM((HALF, NUM_HIDDEN), jnp.float32),
            pltpu.VMEM((B_PER_W,), jnp.float32),
            pltpu.SemaphoreType.DMA,
            pltpu.SemaphoreType.DMA,
        ],
    )
    def sc_kernel(uidx_hbm, iidx_hbm, user_hbm, item_hbm, out_hbm,
                  uidx_v, iidx_v, urows_v, irows_v, out_v,
                  sem_u, sem_i):
        wid = lax.axis_index("s") * NC + lax.axis_index("c")
        base = wid * B_PER_W
        row_iota = lax.iota(jnp.int32, L)

        pltpu.sync_copy(uidx_hbm.at[pl.ds(base, B_PER_W)], uidx_v)
        pltpu.sync_copy(iidx_hbm.at[pl.ds(base, B_PER_W)], iidx_v)

        def half_body(h, carry):
            hbase = h * HALF

            def fetch_body(g, c):
                uvec = uidx_v[pl.ds(hbase + g * L, L)]
                ivec = iidx_v[pl.ds(hbase + g * L, L)]
                for j in range(L):
                    pltpu.async_copy(
                        user_hbm.at[pl.ds(_scalar(uvec, j), 1)],
                        urows_v.at[pl.ds(g * L + j, 1)], sem_u)
                    pltpu.async_copy(
                        item_hbm.at[pl.ds(_scalar(ivec, j), 1)],
                        irows_v.at[pl.ds(g * L + j, 1)], sem_i)
                return c

            lax.fori_loop(0, HALF // L, fetch_body, 0)
            pltpu.make_async_copy(user_hbm.at[pl.ds(0, HALF)], urows_v,
                                  sem_u).wait()
            pltpu.make_async_copy(item_hbm.at[pl.ds(0, HALF)], irows_v,
                                  sem_i).wait()

            def group_body(g, c):
                rows = g * L + row_iota
                acc = jnp.zeros((L,), jnp.float32)
                for col_h in range(NUM_HIDDEN):
                    col = jnp.full((L,), col_h, jnp.int32)
                    u = plsc.load_gather(urows_v, [rows, col])
                    v = plsc.load_gather(irows_v, [rows, col])
                    acc = acc + u * v
                out_v[pl.ds(hbase + g * L, L)] = acc
                return c

            lax.fori_loop(0, HALF // L, group_body, 0)
            return carry

        lax.fori_loop(0, 2, half_body, 0)
        pltpu.sync_copy(out_v, out_hbm.at[pl.ds(base, B_PER_W)])

    return sc_kernel


_SC_KERNEL = _make_sc_kernel()


@jax.jit
def kernel(indices, ratings, user_table, item_table):
    idx = indices.astype(jnp.int32)
    pred = _SC_KERNEL(idx[0], idx[1], user_table, item_table)
    return (pred, ratings)


# speedup vs baseline: 3.6147x; 1.0034x over previous
import functools

import jax
import jax.numpy as jnp
from jax import lax
from jax.experimental import pallas as pl
from jax.experimental.pallas import tpu as pltpu
from jax.experimental.pallas import tpu_sc as plsc

NUM_HIDDEN = 32
BATCH = 16384
NC = 2
NS = 16
NW = NC * NS
B_PER_W = BATCH // NW
QTR = B_PER_W // 4
L = 16


def _scalar(vec, j):
    return jnp.squeeze(lax.slice(vec, (j,), (j + 1,)))


def _make_sc_kernel():
    mesh = plsc.VectorSubcoreMesh(core_axis_name="c", subcore_axis_name="s")

    @functools.partial(
        pl.kernel,
        mesh=mesh,
        compiler_params=pltpu.CompilerParams(needs_layout_passes=False),
        out_type=jax.ShapeDtypeStruct((BATCH,), jnp.float32),
        scratch_types=[
            pltpu.VMEM((B_PER_W,), jnp.int32),
            pltpu.VMEM((B_PER_W,), jnp.int32),
            [pltpu.VMEM((QTR, NUM_HIDDEN), jnp.float32)] * 2,
            [pltpu.VMEM((QTR, NUM_HIDDEN), jnp.float32)] * 2,
            pltpu.VMEM((B_PER_W,), jnp.float32),
            [pltpu.SemaphoreType.DMA] * 2,
            [pltpu.SemaphoreType.DMA] * 2,
        ],
    )
    def sc_kernel(uidx_hbm, iidx_hbm, user_hbm, item_hbm, out_hbm,
                  uidx_v, iidx_v, urows, irows, out_v,
                  sems_u, sems_i):
        wid = lax.axis_index("s") * NC + lax.axis_index("c")
        base = wid * B_PER_W
        row_iota = lax.iota(jnp.int32, L)

        pltpu.sync_copy(uidx_hbm.at[pl.ds(base, B_PER_W)], uidx_v)
        pltpu.sync_copy(iidx_hbm.at[pl.ds(base, B_PER_W)], iidx_v)

        def issue_qtr(q):
            qbase = q * QTR
            b = q % 2

            def fetch_body(g, c):
                uvec = uidx_v[pl.ds(qbase + g * L, L)]
                ivec = iidx_v[pl.ds(qbase + g * L, L)]
                for j in range(L):
                    pltpu.async_copy(
                        user_hbm.at[pl.ds(_scalar(uvec, j), 1)],
                        urows[b].at[pl.ds(g * L + j, 1)], sems_u[b])
                    pltpu.async_copy(
                        item_hbm.at[pl.ds(_scalar(ivec, j), 1)],
                        irows[b].at[pl.ds(g * L + j, 1)], sems_i[b])
                return c

            lax.fori_loop(0, QTR // L, fetch_body, 0)

        def compute_qtr(q):
            qbase = q * QTR
            b = q % 2
            pltpu.make_async_copy(user_hbm.at[pl.ds(0, QTR)], urows[b],
                                  sems_u[b]).wait()
            pltpu.make_async_copy(item_hbm.at[pl.ds(0, QTR)], irows[b],
                                  sems_i[b]).wait()

            def group_body(g, c):
                rows = g * L + row_iota
                acc = jnp.zeros((L,), jnp.float32)
                for col_h in range(NUM_HIDDEN):
                    col = jnp.full((L,), col_h, jnp.int32)
                    u = plsc.load_gather(urows[b], [rows, col])
                    v = plsc.load_gather(irows[b], [rows, col])
                    acc = acc + u * v
                out_v[pl.ds(qbase + g * L, L)] = acc
                return c

            lax.fori_loop(0, QTR // L, group_body, 0)

        issue_qtr(0)
        issue_qtr(1)
        compute_qtr(0)
        issue_qtr(2)
        compute_qtr(1)
        issue_qtr(3)
        compute_qtr(2)
        compute_qtr(3)
        pltpu.sync_copy(out_v, out_hbm.at[pl.ds(base, B_PER_W)])

    return sc_kernel


_SC_KERNEL = _make_sc_kernel()


@jax.jit
def kernel(indices, ratings, user_table, item_table):
    idx = indices.astype(jnp.int32)
    pred = _SC_KERNEL(idx[0], idx[1], user_table, item_table)
    return (pred, ratings)
